# baseline (speedup 1.0000x reference)
import functools

import jax
import jax.numpy as jnp
import numpy as np
from jax.experimental import pallas as pl

_ROWS = 2048
_DIM = 4096
_HALF = _DIM // 2
_BW = 256
_CAP = _BW - 1

_BM = 256
_BNF = 512
_BNI = 512

_INV_N = 1.0 / _DIM


def _trig_tables():
    n = np.arange(_HALF)
    ang = 2.0 * np.pi * ((n[:, None] * n[None, :]) % _DIM) / _DIM
    return np.cos(ang).astype(np.float32), np.sin(ang).astype(np.float32)


def _split_bf16(a):
    hi = a.astype(jnp.bfloat16)
    lo = (a - hi.astype(np.float32)).astype(jnp.bfloat16)
    return hi, lo


_C_TAB, _S_TAB = _trig_tables()
_C_HI, _C_LO = _split_bf16(_C_TAB)
_S_HI, _S_LO = _split_bf16(_S_TAB)


def _split3(x):
    xh = x.astype(jnp.bfloat16)
    xl = (x - xh.astype(jnp.float32)).astype(jnp.bfloat16)
    return xh, xl


def _dot3(xh, xl, t_hi, t_lo):
    f = lambda a, b: jax.lax.dot(a, b, preferred_element_type=jnp.float32)
    return f(xh, t_hi) + (f(xh, t_lo) + f(xl, t_hi))


def _fwd_kernel(xlo_ref, xhr_ref, ch_ref, cl_ref, sh_ref, sl_ref,
                cm_ref, sm_ref):
    xlo = xlo_ref[...]
    xhr = xhr_ref[...]
    xe = xlo + xhr
    xo = xlo - xhr
    x_mid = xhr[:, 0:1]
    eh, el = _split3(xe)
    oh, ol = _split3(xo)
    j = pl.program_id(1)
    kg = jax.lax.broadcasted_iota(jnp.int32, cm_ref.shape, 1) + j * _BNF
    parity = jnp.where(kg % 2 == 1, -2.0, 0.0)
    cm_ref[...] = _dot3(eh, el, ch_ref[...], cl_ref[...]) + x_mid * parity
    ni = jax.lax.broadcasted_iota(jnp.int32, xe.shape, 1)
    sgn_n = jnp.where(ni % 2 == 1, -1.0, 1.0)
    c_nyq = jnp.sum(xe * sgn_n, axis=1, keepdims=True)
    sm = _dot3(oh, ol, sh_ref[...], sl_ref[...])
    sm_ref[...] = jnp.where(kg == 0, c_nyq, sm)


def _select_kernel(cm_ref, sm_ref, cmw_ref, smw_ref):
    cm = cm_ref[...]
    sm = sm_ref[...]
    p = cm * cm + sm * sm
    cols = jax.lax.broadcasted_iota(jnp.int32, p.shape, 1)
    pm = jnp.where(cols == 0, -1.0, p)
    q = sm[:, 0:1] * sm[:, 0:1]

    def body(_, lohi):
        lo, hi = lohi
        mid = lo + (hi - lo) // 2
        t = jax.lax.bitcast_convert_type(mid, jnp.float32)
        cnt = (2 * jnp.sum((pm > t).astype(jnp.int32), axis=1, keepdims=True)
               + (q > t).astype(jnp.int32))
        le = cnt <= _CAP
        return jnp.where(le, lo, mid + 1), jnp.where(le, mid, hi)

    lo0 = jnp.zeros((p.shape[0], 1), jnp.int32)
    hi0 = jnp.full((p.shape[0], 1), jnp.int32(0x7F800000))
    lo, hi = jax.lax.fori_loop(0, 31, body, (lo0, hi0))
    tau = jax.lax.bitcast_convert_type(hi, jnp.float32)

    full = pm > tau
    w = 2.0 * full.astype(jnp.float32)
    nyq_gt = q > tau
    used = (2 * jnp.sum(full.astype(jnp.int32), axis=1, keepdims=True)
            + nyq_gt.astype(jnp.int32))
    spare = used < _CAP
    eq = pm == tau
    nyq_eq = q == tau
    eq_idx = jnp.where(eq, cols, 2 * _DIM)
    min_pair = jnp.min(eq_idx, axis=1, keepdims=True)
    min_k = jnp.minimum(min_pair, jnp.where(nyq_eq, _HALF, 2 * _DIM))
    w = w + (spare & eq & (cols == min_k)).astype(jnp.float32)
    w_nyq = nyq_gt.astype(jnp.float32) + (
        spare & nyq_eq & (min_k == _HALF)).astype(jnp.float32)

    wc = jnp.where(cols == 0, 1.0, w)
    ws = jnp.where(cols == 0, w_nyq, w)
    cmw_ref[...] = (cm * wc).astype(jnp.bfloat16)
    smw_ref[...] = (sm * ws).astype(jnp.bfloat16)


def _inv_kernel(cmw_ref, smw_ref, ch_ref, sh_ref, a_ref, b_ref):
    cmw = cmw_ref[...]
    smw = smw_ref[...]
    dims = (((1,), (1,)), ((), ()))
    e = jax.lax.dot_general(cmw, ch_ref[...], dims,
                            preferred_element_type=jnp.float32)
    o = jax.lax.dot_general(smw, sh_ref[...], dims,
                            preferred_element_type=jnp.float32)
    j = pl.program_id(1)
    ng = jax.lax.broadcasted_iota(jnp.int32, a_ref.shape, 1) + j * _BNI
    sgn_n = jnp.where(ng % 2 == 1, -1.0, 1.0)
    smw0 = smw[:, 0:1].astype(jnp.float32)
    e = e + smw0 * sgn_n
    a_ref[...] = (e + o) * _INV_N
    b = (e - o) * _INV_N
    ki = jax.lax.broadcasted_iota(jnp.int32, cmw.shape, 1)
    sgn_k = jnp.where(ki % 2 == 1, -1.0, 1.0)
    out_mid = (jnp.sum(cmw.astype(jnp.float32) * sgn_k, axis=1,
                       keepdims=True) + smw0) * _INV_N
    b_ref[...] = jnp.where(ng == 0, out_mid, b)


@functools.partial(jax.jit)
def kernel(gradient):
    x = gradient.astype(jnp.float32)
    c_hi, c_lo = jnp.asarray(_C_HI), jnp.asarray(_C_LO)
    s_hi, s_lo = jnp.asarray(_S_HI), jnp.asarray(_S_LO)

    x_lo = x[:, :_HALF]
    x_hr = jnp.concatenate(
        [x[:, _HALF:_HALF + 1], jnp.flip(x[:, _HALF + 1:], axis=1)], axis=1)

    fwd = pl.pallas_call(
        _fwd_kernel,
        grid=(_ROWS // _BM, _HALF // _BNF),
        in_specs=[
            pl.BlockSpec((_BM, _HALF), lambda i, j: (i, 0)),
            pl.BlockSpec((_BM, _HALF), lambda i, j: (i, 0)),
            pl.BlockSpec((_HALF, _BNF), lambda i, j: (0, j)),
            pl.BlockSpec((_HALF, _BNF), lambda i, j: (0, j)),
            pl.BlockSpec((_HALF, _BNF), lambda i, j: (0, j)),
            pl.BlockSpec((_HALF, _BNF), lambda i, j: (0, j)),
        ],
        out_specs=[
            pl.BlockSpec((_BM, _BNF), lambda i, j: (i, j)),
            pl.BlockSpec((_BM, _BNF), lambda i, j: (i, j)),
        ],
        out_shape=[
            jax.ShapeDtypeStruct((_ROWS, _HALF), jnp.float32),
            jax.ShapeDtypeStruct((_ROWS, _HALF), jnp.float32),
        ],
    )
    cm, sm = fwd(x_lo, x_hr, c_hi, c_lo, s_hi, s_lo)

    sel = pl.pallas_call(
        _select_kernel,
        grid=(_ROWS // _BM,),
        in_specs=[
            pl.BlockSpec((_BM, _HALF), lambda i: (i, 0)),
            pl.BlockSpec((_BM, _HALF), lambda i: (i, 0)),
        ],
        out_specs=[
            pl.BlockSpec((_BM, _HALF), lambda i: (i, 0)),
            pl.BlockSpec((_BM, _HALF), lambda i: (i, 0)),
        ],
        out_shape=[
            jax.ShapeDtypeStruct((_ROWS, _HALF), jnp.bfloat16),
            jax.ShapeDtypeStruct((_ROWS, _HALF), jnp.bfloat16),
        ],
    )
    cmw, smw = sel(cm, sm)

    inv = pl.pallas_call(
        _inv_kernel,
        grid=(_ROWS // _BM, _HALF // _BNI),
        in_specs=[
            pl.BlockSpec((_BM, _HALF), lambda i, j: (i, 0)),
            pl.BlockSpec((_BM, _HALF), lambda i, j: (i, 0)),
            pl.BlockSpec((_BNI, _HALF), lambda i, j: (j, 0)),
            pl.BlockSpec((_BNI, _HALF), lambda i, j: (j, 0)),
        ],
        out_specs=[
            pl.BlockSpec((_BM, _BNI), lambda i, j: (i, j)),
            pl.BlockSpec((_BM, _BNI), lambda i, j: (i, j)),
        ],
        out_shape=[
            jax.ShapeDtypeStruct((_ROWS, _HALF), jnp.float32),
            jax.ShapeDtypeStruct((_ROWS, _HALF), jnp.float32),
        ],
    )
    a, b = inv(cmw, smw, c_hi, s_hi)

    return jnp.concatenate([a, b[:, 0:1], jnp.flip(b[:, 1:], axis=1)], axis=1)


# speedup vs baseline: 1.0403x; 1.0403x over previous
import functools

import jax
import jax.numpy as jnp
import numpy as np
from jax.experimental import pallas as pl

_ROWS = 2048
_DIM = 4096
_HALF = _DIM // 2
_BW = 256
_CAP = _BW - 1

_BM = 256
_BNF = 1024
_BNI = 1024

_INV_N = 1.0 / _DIM


def _trig_tables():
    n = np.arange(_HALF)
    ang = 2.0 * np.pi * ((n[:, None] * n[None, :]) % _DIM) / _DIM
    return np.cos(ang).astype(np.float32), np.sin(ang).astype(np.float32)


def _split_bf16(a):
    hi = a.astype(jnp.bfloat16)
    lo = (a - hi.astype(np.float32)).astype(jnp.bfloat16)
    return hi, lo


_C_TAB, _S_TAB = _trig_tables()
_C_HI, _C_LO = _split_bf16(_C_TAB)
_S_HI, _S_LO = _split_bf16(_S_TAB)


def _split3(x):
    xh = x.astype(jnp.bfloat16)
    xl = (x - xh.astype(jnp.float32)).astype(jnp.bfloat16)
    return xh, xl


def _dot3(xh, xl, t_hi, t_lo):
    f = lambda a, b: jax.lax.dot(a, b, preferred_element_type=jnp.float32)
    return f(xh, t_hi) + (f(xh, t_lo) + f(xl, t_hi))


def _fwd_kernel(xlo_ref, xhr_ref, ch_ref, cl_ref, sh_ref, sl_ref,
                cm_ref, sm_ref):
    xlo = xlo_ref[...]
    xhr = xhr_ref[...]
    xe = xlo + xhr
    xo = xlo - xhr
    x_mid = xhr[:, 0:1]
    eh, el = _split3(xe)
    oh, ol = _split3(xo)
    j = pl.program_id(0)
    kg = jax.lax.broadcasted_iota(jnp.int32, cm_ref.shape, 1) + j * _BNF
    parity = jnp.where(kg % 2 == 1, -2.0, 0.0)
    cm_ref[...] = _dot3(eh, el, ch_ref[...], cl_ref[...]) + x_mid * parity
    ni = jax.lax.broadcasted_iota(jnp.int32, xe.shape, 1)
    sgn_n = jnp.where(ni % 2 == 1, -1.0, 1.0)
    c_nyq = jnp.sum(xe * sgn_n, axis=1, keepdims=True)
    sm = _dot3(oh, ol, sh_ref[...], sl_ref[...])
    sm_ref[...] = jnp.where(kg == 0, c_nyq, sm)


def _select_kernel(cm_ref, sm_ref, cmw_ref, smw_ref):
    cm = cm_ref[...]
    sm = sm_ref[...]
    p = cm * cm + sm * sm
    cols = jax.lax.broadcasted_iota(jnp.int32, p.shape, 1)
    pm = jnp.where(cols == 0, -1.0, p)
    q = sm[:, 0:1] * sm[:, 0:1]

    def body(_, lohi):
        lo, hi = lohi
        mid = lo + (hi - lo) // 2
        t = jax.lax.bitcast_convert_type(mid, jnp.float32)
        cnt = (2 * jnp.sum((pm > t).astype(jnp.int32), axis=1, keepdims=True)
               + (q > t).astype(jnp.int32))
        le = cnt <= _CAP
        return jnp.where(le, lo, mid + 1), jnp.where(le, mid, hi)

    lo0 = jnp.zeros((p.shape[0], 1), jnp.int32)
    hi0 = jnp.full((p.shape[0], 1), jnp.int32(0x7F800000))
    lo, hi = jax.lax.fori_loop(0, 31, body, (lo0, hi0))
    tau = jax.lax.bitcast_convert_type(hi, jnp.float32)

    full = pm > tau
    w = 2.0 * full.astype(jnp.float32)
    nyq_gt = q > tau
    used = (2 * jnp.sum(full.astype(jnp.int32), axis=1, keepdims=True)
            + nyq_gt.astype(jnp.int32))
    spare = used < _CAP
    eq = pm == tau
    nyq_eq = q == tau
    eq_idx = jnp.where(eq, cols, 2 * _DIM)
    min_pair = jnp.min(eq_idx, axis=1, keepdims=True)
    min_k = jnp.minimum(min_pair, jnp.where(nyq_eq, _HALF, 2 * _DIM))
    w = w + (spare & eq & (cols == min_k)).astype(jnp.float32)
    w_nyq = nyq_gt.astype(jnp.float32) + (
        spare & nyq_eq & (min_k == _HALF)).astype(jnp.float32)

    wc = jnp.where(cols == 0, 1.0, w)
    ws = jnp.where(cols == 0, w_nyq, w)
    cmw_ref[...] = (cm * wc).astype(jnp.bfloat16)
    smw_ref[...] = (sm * ws).astype(jnp.bfloat16)


def _inv_kernel(cmw_ref, smw_ref, ch_ref, sh_ref, a_ref, b_ref):
    cmw = cmw_ref[...]
    smw = smw_ref[...]
    dims = (((1,), (1,)), ((), ()))
    e = jax.lax.dot_general(cmw, ch_ref[...], dims,
                            preferred_element_type=jnp.float32)
    o = jax.lax.dot_general(smw, sh_ref[...], dims,
                            preferred_element_type=jnp.float32)
    j = pl.program_id(0)
    ng = jax.lax.broadcasted_iota(jnp.int32, a_ref.shape, 1) + j * _BNI
    sgn_n = jnp.where(ng % 2 == 1, -1.0, 1.0)
    smw0 = smw[:, 0:1].astype(jnp.float32)
    e = e + smw0 * sgn_n
    a_ref[...] = (e + o) * _INV_N
    b = (e - o) * _INV_N
    ki = jax.lax.broadcasted_iota(jnp.int32, cmw.shape, 1)
    sgn_k = jnp.where(ki % 2 == 1, -1.0, 1.0)
    out_mid = (jnp.sum(cmw.astype(jnp.float32) * sgn_k, axis=1,
                       keepdims=True) + smw0) * _INV_N
    b_ref[...] = jnp.where(ng == 0, out_mid, b)


@functools.partial(jax.jit)
def kernel(gradient):
    x = gradient.astype(jnp.float32)
    c_hi, c_lo = jnp.asarray(_C_HI), jnp.asarray(_C_LO)
    s_hi, s_lo = jnp.asarray(_S_HI), jnp.asarray(_S_LO)

    x_lo = x[:, :_HALF]
    x_hr = jnp.concatenate(
        [x[:, _HALF:_HALF + 1], jnp.flip(x[:, _HALF + 1:], axis=1)], axis=1)

    fwd = pl.pallas_call(
        _fwd_kernel,
        grid=(_HALF // _BNF, _ROWS // _BM),
        in_specs=[
            pl.BlockSpec((_BM, _HALF), lambda j, i: (i, 0)),
            pl.BlockSpec((_BM, _HALF), lambda j, i: (i, 0)),
            pl.BlockSpec((_HALF, _BNF), lambda j, i: (0, j)),
            pl.BlockSpec((_HALF, _BNF), lambda j, i: (0, j)),
            pl.BlockSpec((_HALF, _BNF), lambda j, i: (0, j)),
            pl.BlockSpec((_HALF, _BNF), lambda j, i: (0, j)),
        ],
        out_specs=[
            pl.BlockSpec((_BM, _BNF), lambda j, i: (i, j)),
            pl.BlockSpec((_BM, _BNF), lambda j, i: (i, j)),
        ],
        out_shape=[
            jax.ShapeDtypeStruct((_ROWS, _HALF), jnp.float32),
            jax.ShapeDtypeStruct((_ROWS, _HALF), jnp.float32),
        ],
    )
    cm, sm = fwd(x_lo, x_hr, c_hi, c_lo, s_hi, s_lo)

    sel = pl.pallas_call(
        _select_kernel,
        grid=(_ROWS // _BM,),
        in_specs=[
            pl.BlockSpec((_BM, _HALF), lambda i: (i, 0)),
            pl.BlockSpec((_BM, _HALF), lambda i: (i, 0)),
        ],
        out_specs=[
            pl.BlockSpec((_BM, _HALF), lambda i: (i, 0)),
            pl.BlockSpec((_BM, _HALF), lambda i: (i, 0)),
        ],
        out_shape=[
            jax.ShapeDtypeStruct((_ROWS, _HALF), jnp.bfloat16),
            jax.ShapeDtypeStruct((_ROWS, _HALF), jnp.bfloat16),
        ],
    )
    cmw, smw = sel(cm, sm)

    inv = pl.pallas_call(
        _inv_kernel,
        grid=(_HALF // _BNI, _ROWS // _BM),
        in_specs=[
            pl.BlockSpec((_BM, _HALF), lambda j, i: (i, 0)),
            pl.BlockSpec((_BM, _HALF), lambda j, i: (i, 0)),
            pl.BlockSpec((_BNI, _HALF), lambda j, i: (j, 0)),
            pl.BlockSpec((_BNI, _HALF), lambda j, i: (j, 0)),
        ],
        out_specs=[
            pl.BlockSpec((_BM, _BNI), lambda j, i: (i, j)),
            pl.BlockSpec((_BM, _BNI), lambda j, i: (i, j)),
        ],
        out_shape=[
            jax.ShapeDtypeStruct((_ROWS, _HALF), jnp.float32),
            jax.ShapeDtypeStruct((_ROWS, _HALF), jnp.float32),
        ],
    )
    a, b = inv(cmw, smw, c_hi, s_hi)

    return jnp.concatenate([a, b[:, 0:1], jnp.flip(b[:, 1:], axis=1)], axis=1)


# speedup vs baseline: 1.5177x; 1.4590x over previous
import functools

import jax
import jax.numpy as jnp
import numpy as np
from jax.experimental import pallas as pl
from jax.experimental.pallas import tpu as pltpu

_ROWS = 2048
_DIM = 4096
_HALF = _DIM // 2
_BW = 256
_CAP = _BW - 1

_BM = 256
_BNF = 512

_INV_N = 1.0 / _DIM


def _trig_tables():
    n = np.arange(_HALF)
    ang = 2.0 * np.pi * ((n[:, None] * n[None, :]) % _DIM) / _DIM
    return np.cos(ang).astype(np.float32), np.sin(ang).astype(np.float32)


def _split_bf16(a):
    hi = a.astype(jnp.bfloat16)
    lo = (a - hi.astype(np.float32)).astype(jnp.bfloat16)
    return hi, lo


_C_TAB, _S_TAB = _trig_tables()
_C_HI, _C_LO = _split_bf16(_C_TAB)
_S_HI, _S_LO = _split_bf16(_S_TAB)


def _perm_table():
    j2 = np.zeros((_HALF, _HALF), dtype=np.float32)
    n = np.arange(_HALF)
    j2[(_HALF - n) % _HALF, n] = 1.0
    return j2.astype(jnp.bfloat16)


_J2 = _perm_table()


def _split3(x):
    xh = x.astype(jnp.bfloat16)
    xl = (x - xh.astype(jnp.float32)).astype(jnp.bfloat16)
    return xh, xl


def _dot3(xh, xl, t_hi, t_lo):
    f = lambda a, b: jax.lax.dot(a, b, preferred_element_type=jnp.float32)
    return f(xh, t_hi) + (f(xh, t_lo) + f(xl, t_hi))


def _fwd_kernel(x_ref, j2_ref, ch_ref, cl_ref, sh_ref, sl_ref,
                cm_ref, sm_ref):
    x = x_ref[...]
    xa = x[:, :_HALF]
    f = lambda a, b: jax.lax.dot(a, b, preferred_element_type=jnp.float32)
    xbh, xbl = _split3(x[:, _HALF:])
    j2 = j2_ref[...]
    xhr = f(xbh, j2) + f(xbl, j2)
    xe = xa + xhr
    xo = xa - xhr
    x_mid = xhr[:, 0:1]
    eh, el = _split3(xe)
    oh, ol = _split3(xo)
    j = pl.program_id(0)
    kg = jax.lax.broadcasted_iota(jnp.int32, cm_ref.shape, 1) + j * _BNF
    parity = jnp.where(kg % 2 == 1, -2.0, 0.0)
    cm_ref[...] = _dot3(eh, el, ch_ref[...], cl_ref[...]) + x_mid * parity
    ni = jax.lax.broadcasted_iota(jnp.int32, xe.shape, 1)
    sgn_n = jnp.where(ni % 2 == 1, -1.0, 1.0)
    c_nyq = jnp.sum(xe * sgn_n, axis=1, keepdims=True)
    sm = _dot3(oh, ol, sh_ref[...], sl_ref[...])
    sm_ref[...] = jnp.where(kg == 0, c_nyq, sm)


def _select_kernel(cm_ref, sm_ref, cmw_ref, smw_ref):
    cm = cm_ref[...]
    sm = sm_ref[...]
    p = cm * cm + sm * sm
    cols = jax.lax.broadcasted_iota(jnp.int32, p.shape, 1)
    pm = jnp.where(cols == 0, -1.0, p)
    q = sm[:, 0:1] * sm[:, 0:1]

    def body(_, lohi):
        lo, hi = lohi
        mid = lo + (hi - lo) // 2
        t = jax.lax.bitcast_convert_type(mid, jnp.float32)
        cnt = (2 * jnp.sum((pm > t).astype(jnp.int32), axis=1, keepdims=True)
               + (q > t).astype(jnp.int32))
        le = cnt <= _CAP
        return jnp.where(le, lo, mid + 1), jnp.where(le, mid, hi)

    lo0 = jnp.zeros((p.shape[0], 1), jnp.int32)
    hi0 = jnp.full((p.shape[0], 1), jnp.int32(0x7F800000))
    lo, hi = jax.lax.fori_loop(0, 31, body, (lo0, hi0))
    tau = jax.lax.bitcast_convert_type(hi, jnp.float32)

    full = pm > tau
    w = 2.0 * full.astype(jnp.float32)
    nyq_gt = q > tau
    used = (2 * jnp.sum(full.astype(jnp.int32), axis=1, keepdims=True)
            + nyq_gt.astype(jnp.int32))
    spare = used < _CAP
    eq = pm == tau
    nyq_eq = q == tau
    eq_idx = jnp.where(eq, cols, 2 * _DIM)
    min_pair = jnp.min(eq_idx, axis=1, keepdims=True)
    min_k = jnp.minimum(min_pair, jnp.where(nyq_eq, _HALF, 2 * _DIM))
    w = w + (spare & eq & (cols == min_k)).astype(jnp.float32)
    w_nyq = nyq_gt.astype(jnp.float32) + (
        spare & nyq_eq & (min_k == _HALF)).astype(jnp.float32)

    wc = jnp.where(cols == 0, 1.0, w)
    ws = jnp.where(cols == 0, w_nyq, w)
    cmw_ref[...] = (cm * wc).astype(jnp.bfloat16)
    smw_ref[...] = (sm * ws).astype(jnp.bfloat16)


def _inv_kernel(cmw_ref, smw_ref, j2_ref, ch_ref, sh_ref, o_ref):
    cmw = cmw_ref[...]
    smw = smw_ref[...]
    dims = (((1,), (1,)), ((), ()))
    e = jax.lax.dot_general(cmw, ch_ref[...], dims,
                            preferred_element_type=jnp.float32)
    o = jax.lax.dot_general(smw, sh_ref[...], dims,
                            preferred_element_type=jnp.float32)
    ng = jax.lax.broadcasted_iota(jnp.int32, e.shape, 1)
    sgn_n = jnp.where(ng % 2 == 1, -1.0, 1.0)
    smw0 = smw[:, 0:1].astype(jnp.float32)
    e = e + smw0 * sgn_n
    a = (e + o) * _INV_N
    b = (e - o) * _INV_N
    ki = jax.lax.broadcasted_iota(jnp.int32, cmw.shape, 1)
    sgn_k = jnp.where(ki % 2 == 1, -1.0, 1.0)
    out_mid = (jnp.sum(cmw.astype(jnp.float32) * sgn_k, axis=1,
                       keepdims=True) + smw0) * _INV_N
    b = jnp.where(ng == 0, out_mid, b)
    bh2, bl2 = _split3(b)
    f = lambda u, v: jax.lax.dot(u, v, preferred_element_type=jnp.float32)
    j2 = j2_ref[...]
    w_hi = f(bh2, j2) + f(bl2, j2)
    o_ref[:, :_HALF] = a
    o_ref[:, _HALF:] = w_hi


@functools.partial(jax.jit)
def kernel(gradient):
    x = gradient.astype(jnp.float32)
    c_hi, c_lo = jnp.asarray(_C_HI), jnp.asarray(_C_LO)
    s_hi, s_lo = jnp.asarray(_S_HI), jnp.asarray(_S_LO)
    j2 = jnp.asarray(_J2)

    fwd = pl.pallas_call(
        _fwd_kernel,
        grid=(_HALF // _BNF, _ROWS // _BM),
        in_specs=[
            pl.BlockSpec((_BM, _DIM), lambda j, i: (i, 0)),
            pl.BlockSpec((_HALF, _HALF), lambda j, i: (0, 0)),
            pl.BlockSpec((_HALF, _BNF), lambda j, i: (0, j)),
            pl.BlockSpec((_HALF, _BNF), lambda j, i: (0, j)),
            pl.BlockSpec((_HALF, _BNF), lambda j, i: (0, j)),
            pl.BlockSpec((_HALF, _BNF), lambda j, i: (0, j)),
        ],
        out_specs=[
            pl.BlockSpec((_BM, _BNF), lambda j, i: (i, j)),
            pl.BlockSpec((_BM, _BNF), lambda j, i: (i, j)),
        ],
        out_shape=[
            jax.ShapeDtypeStruct((_ROWS, _HALF), jnp.float32),
            jax.ShapeDtypeStruct((_ROWS, _HALF), jnp.float32),
        ],
    )
    cm, sm = fwd(x, j2, c_hi, c_lo, s_hi, s_lo)

    sel = pl.pallas_call(
        _select_kernel,
        grid=(_ROWS // _BM,),
        in_specs=[
            pl.BlockSpec((_BM, _HALF), lambda i: (i, 0)),
            pl.BlockSpec((_BM, _HALF), lambda i: (i, 0)),
        ],
        out_specs=[
            pl.BlockSpec((_BM, _HALF), lambda i: (i, 0)),
            pl.BlockSpec((_BM, _HALF), lambda i: (i, 0)),
        ],
        out_shape=[
            jax.ShapeDtypeStruct((_ROWS, _HALF), jnp.bfloat16),
            jax.ShapeDtypeStruct((_ROWS, _HALF), jnp.bfloat16),
        ],
    )
    cmw, smw = sel(cm, sm)

    inv = pl.pallas_call(
        _inv_kernel,
        grid=(_ROWS // _BM,),
        in_specs=[
            pl.BlockSpec((_BM, _HALF), lambda i: (i, 0)),
            pl.BlockSpec((_BM, _HALF), lambda i: (i, 0)),
            pl.BlockSpec((_HALF, _HALF), lambda i: (0, 0)),
            pl.BlockSpec((_HALF, _HALF), lambda i: (0, 0)),
            pl.BlockSpec((_HALF, _HALF), lambda i: (0, 0)),
        ],
        out_specs=pl.BlockSpec((_BM, _DIM), lambda i: (i, 0)),
        out_shape=jax.ShapeDtypeStruct((_ROWS, _DIM), jnp.float32),
    )
    return inv(cmw, smw, j2, c_hi, s_hi)
